# baseline (speedup 1.0000x reference)
import jax
import jax.numpy as jnp
from jax import lax
from jax.experimental import pallas as pl
from jax.experimental.pallas import tpu as pltpu
from jax.experimental.pallas import tpu_sc as plsc

N_ROWS = 16384
C_TRUNC = 1024
B_TRUNC = 2048
C_OUT = 512
B_OUT = 1024
L = 16
NC = 2
NS = 16
NW = NC * NS
ROWS_PER_W = N_ROWS // NW
R = 16
N_CHUNKS = ROWS_PER_W // R


def _gather_phase(in_buf, out_buf, idx_buf, n_groups):
    for j in range(n_groups):
        idx = idx_buf[pl.ds(j * L, L)]

        def row_body(r, carry, idx=idx):
            rows = jnp.full((L,), r, dtype=jnp.int32)
            vals = plsc.load_gather(in_buf, [rows, idx])
            out_buf[r, pl.ds(j * L, L)] = vals
            return carry

        lax.fori_loop(0, R, row_body, 0)


def _body(cont_hbm, bin_hbm, cmask_hbm, bmask_hbm, out_c_hbm, out_b_hbm,
          cidx, bidx, cin, cout, bin_in, bout):
    wid = lax.axis_index("c") * NS + lax.axis_index("s")
    base0 = wid * ROWS_PER_W
    pltpu.sync_copy(cmask_hbm, cidx)
    pltpu.sync_copy(bmask_hbm, bidx)

    def chunk_body(chunk, carry):
        base = base0 + chunk * R
        pltpu.sync_copy(cont_hbm.at[pl.ds(base, R), pl.ds(0, C_TRUNC)], cin)
        _gather_phase(cin, cout, cidx, C_OUT // L)
        pltpu.sync_copy(cout, out_c_hbm.at[pl.ds(base, R), :])

        pltpu.sync_copy(bin_hbm.at[pl.ds(base, R), pl.ds(0, B_TRUNC)], bin_in)
        _gather_phase(bin_in, bout, bidx, B_OUT // L)
        pltpu.sync_copy(bout, out_b_hbm.at[pl.ds(base, R), :])
        return carry

    lax.fori_loop(0, N_CHUNKS, chunk_body, 0)


def kernel(continuous, binary, continuous_mask, binary_mask):
    mesh = plsc.VectorSubcoreMesh(core_axis_name="c", subcore_axis_name="s")
    k = pl.kernel(
        _body,
        out_type=(
            jax.ShapeDtypeStruct((N_ROWS, C_OUT), jnp.float32),
            jax.ShapeDtypeStruct((N_ROWS, B_OUT), jnp.float32),
        ),
        mesh=mesh,
        compiler_params=pltpu.CompilerParams(
            use_tc_tiling_on_sc=False, needs_layout_passes=False),
        scratch_types=[
            pltpu.VMEM((C_OUT,), jnp.int32),
            pltpu.VMEM((B_OUT,), jnp.int32),
            pltpu.VMEM((R, C_TRUNC), jnp.float32),
            pltpu.VMEM((R, C_OUT), jnp.float32),
            pltpu.VMEM((R, B_TRUNC), jnp.float32),
            pltpu.VMEM((R, B_OUT), jnp.float32),
        ],
    )
    return k(continuous, binary, continuous_mask, binary_mask)


# speedup vs baseline: 1.1634x; 1.1634x over previous
import jax
import jax.numpy as jnp
from jax import lax
from jax.experimental import pallas as pl
from jax.experimental.pallas import tpu as pltpu
from jax.experimental.pallas import tpu_sc as plsc

N_ROWS = 16384
C_TRUNC = 1024
B_TRUNC = 2048
C_OUT = 512
B_OUT = 1024
L = 16
NC = 2
NS = 16
NW = NC * NS
ROWS_PER_W = N_ROWS // NW
R = 16
N_CHUNKS = ROWS_PER_W // R


def _gather_phase(in_buf, out_buf, idx_buf, n_groups, stride):
    in_row = in_buf.at[0]
    for j in range(n_groups):
        idx0 = idx_buf[pl.ds(j * L, L)]

        def row_body(r, fidx, j=j):
            vals = plsc.load_gather(in_row, [fidx])
            out_buf[r, pl.ds(j * L, L)] = vals
            return fidx + stride

        plsc.parallel_loop(0, R, 1, unroll=4, carry=idx0)(row_body)


def _body(cont_hbm, bin_hbm, cmask_hbm, bmask_hbm, out_c_hbm, out_b_hbm,
          cidx, bidx, cin, cout, bin_in, bout):
    wid = lax.axis_index("c") * NS + lax.axis_index("s")
    base0 = wid * ROWS_PER_W
    pltpu.sync_copy(cmask_hbm, cidx)
    pltpu.sync_copy(bmask_hbm, bidx)

    def chunk_body(chunk, carry):
        base = base0 + chunk * R
        pltpu.sync_copy(cont_hbm.at[pl.ds(base, R), pl.ds(0, C_TRUNC)], cin)
        _gather_phase(cin, cout, cidx, C_OUT // L, C_TRUNC)
        pltpu.sync_copy(cout, out_c_hbm.at[pl.ds(base, R), :])

        pltpu.sync_copy(bin_hbm.at[pl.ds(base, R), pl.ds(0, B_TRUNC)], bin_in)
        _gather_phase(bin_in, bout, bidx, B_OUT // L, B_TRUNC)
        pltpu.sync_copy(bout, out_b_hbm.at[pl.ds(base, R), :])
        return carry

    lax.fori_loop(0, N_CHUNKS, chunk_body, 0)


def kernel(continuous, binary, continuous_mask, binary_mask):
    mesh = plsc.VectorSubcoreMesh(core_axis_name="c", subcore_axis_name="s")
    k = pl.kernel(
        _body,
        out_type=(
            jax.ShapeDtypeStruct((N_ROWS, C_OUT), jnp.float32),
            jax.ShapeDtypeStruct((N_ROWS, B_OUT), jnp.float32),
        ),
        mesh=mesh,
        compiler_params=pltpu.CompilerParams(
            use_tc_tiling_on_sc=False, needs_layout_passes=False),
        scratch_types=[
            pltpu.VMEM((C_OUT,), jnp.int32),
            pltpu.VMEM((B_OUT,), jnp.int32),
            pltpu.VMEM((R, C_TRUNC), jnp.float32),
            pltpu.VMEM((R, C_OUT), jnp.float32),
            pltpu.VMEM((R, B_TRUNC), jnp.float32),
            pltpu.VMEM((R, B_OUT), jnp.float32),
        ],
    )
    return k(continuous, binary, continuous_mask, binary_mask)


# speedup vs baseline: 1.3195x; 1.1342x over previous
import jax
import jax.numpy as jnp
from jax import lax
from jax.experimental import pallas as pl
from jax.experimental.pallas import tpu as pltpu
from jax.experimental.pallas import tpu_sc as plsc

N_ROWS = 16384
C_TRUNC = 1024
B_TRUNC = 2048
C_OUT = 512
B_OUT = 1024
L = 16
NC = 2
NS = 16
NW = NC * NS
ROWS_PER_W = N_ROWS // NW
R = 8
N_CHUNKS = ROWS_PER_W // R


def _gather_phase(in_buf, out_buf, idx_buf, n_groups, stride):
    in_row = in_buf.at[0]
    for j in range(n_groups):
        idx0 = idx_buf[pl.ds(j * L, L)]

        def row_body(r, fidx, j=j):
            vals = plsc.load_gather(in_row, [fidx])
            out_buf[r, pl.ds(j * L, L)] = vals
            return fidx + stride

        plsc.parallel_loop(0, R, 1, unroll=4, carry=idx0)(row_body)


def _body(cont_hbm, bin_hbm, cmask_hbm, bmask_hbm, out_c_hbm, out_b_hbm,
          cidx, bidx,
          cin0, cin1, cout0, cout1, bin0, bin1, bout0, bout1,
          sci0, sci1, sbi0, sbi1, sco0, sco1, sbo0, sbo1):
    cin = (cin0, cin1)
    cout = (cout0, cout1)
    bins = (bin0, bin1)
    bout = (bout0, bout1)
    sci = (sci0, sci1)
    sbi = (sbi0, sbi1)
    sco = (sco0, sco1)
    sbo = (sbo0, sbo1)

    wid = lax.axis_index("c") * NS + lax.axis_index("s")
    base0 = wid * ROWS_PER_W
    pltpu.sync_copy(cmask_hbm, cidx)
    pltpu.sync_copy(bmask_hbm, bidx)

    def start_in(chunk, b):
        base = base0 + chunk * R
        pltpu.async_copy(
            cont_hbm.at[pl.ds(base, R), pl.ds(0, C_TRUNC)], cin[b], sci[b])
        pltpu.async_copy(
            bin_hbm.at[pl.ds(base, R), pl.ds(0, B_TRUNC)], bins[b], sbi[b])

    start_in(0, 0)

    def pair_body(g, carry):
        for b in (0, 1):
            chunk = 2 * g + b
            base = base0 + chunk * R

            @pl.when(chunk + 1 < N_CHUNKS)
            def _(b=b, chunk=chunk):
                start_in(chunk + 1, 1 - b)

            @pl.when(chunk >= 2)
            def _(b=b, chunk=chunk):
                pb = base0 + (chunk - 2) * R
                pltpu.make_async_copy(
                    cout[b], out_c_hbm.at[pl.ds(pb, R), :], sco[b]).wait()
                pltpu.make_async_copy(
                    bout[b], out_b_hbm.at[pl.ds(pb, R), :], sbo[b]).wait()

            pltpu.make_async_copy(
                cont_hbm.at[pl.ds(base, R), pl.ds(0, C_TRUNC)],
                cin[b], sci[b]).wait()
            _gather_phase(cin[b], cout[b], cidx, C_OUT // L, C_TRUNC)
            pltpu.async_copy(cout[b], out_c_hbm.at[pl.ds(base, R), :], sco[b])

            pltpu.make_async_copy(
                bin_hbm.at[pl.ds(base, R), pl.ds(0, B_TRUNC)],
                bins[b], sbi[b]).wait()
            _gather_phase(bins[b], bout[b], bidx, B_OUT // L, B_TRUNC)
            pltpu.async_copy(bout[b], out_b_hbm.at[pl.ds(base, R), :], sbo[b])
        return carry

    lax.fori_loop(0, N_CHUNKS // 2, pair_body, 0)

    for b in (0, 1):
        pb = base0 + (N_CHUNKS - 2 + b) * R
        pltpu.make_async_copy(
            cout[b], out_c_hbm.at[pl.ds(pb, R), :], sco[b]).wait()
        pltpu.make_async_copy(
            bout[b], out_b_hbm.at[pl.ds(pb, R), :], sbo[b]).wait()


def kernel(continuous, binary, continuous_mask, binary_mask):
    mesh = plsc.VectorSubcoreMesh(core_axis_name="c", subcore_axis_name="s")
    k = pl.kernel(
        _body,
        out_type=(
            jax.ShapeDtypeStruct((N_ROWS, C_OUT), jnp.float32),
            jax.ShapeDtypeStruct((N_ROWS, B_OUT), jnp.float32),
        ),
        mesh=mesh,
        compiler_params=pltpu.CompilerParams(
            use_tc_tiling_on_sc=False, needs_layout_passes=False),
        scratch_types=[
            pltpu.VMEM((C_OUT,), jnp.int32),
            pltpu.VMEM((B_OUT,), jnp.int32),
            pltpu.VMEM((R, C_TRUNC), jnp.float32),
            pltpu.VMEM((R, C_TRUNC), jnp.float32),
            pltpu.VMEM((R, C_OUT), jnp.float32),
            pltpu.VMEM((R, C_OUT), jnp.float32),
            pltpu.VMEM((R, B_TRUNC), jnp.float32),
            pltpu.VMEM((R, B_TRUNC), jnp.float32),
            pltpu.VMEM((R, B_OUT), jnp.float32),
            pltpu.VMEM((R, B_OUT), jnp.float32),
            pltpu.SemaphoreType.DMA,
            pltpu.SemaphoreType.DMA,
            pltpu.SemaphoreType.DMA,
            pltpu.SemaphoreType.DMA,
            pltpu.SemaphoreType.DMA,
            pltpu.SemaphoreType.DMA,
            pltpu.SemaphoreType.DMA,
            pltpu.SemaphoreType.DMA,
        ],
    )
    return k(continuous, binary, continuous_mask, binary_mask)


# speedup vs baseline: 2.7971x; 2.1198x over previous
import jax
import jax.numpy as jnp
from jax import lax
from jax.experimental import pallas as pl
from jax.experimental.pallas import tpu as pltpu
from jax.experimental.pallas import tpu_sc as plsc

N_ROWS = 16384
C_TRUNC = 1024
B_TRUNC = 2048
C_OUT = 512
B_OUT = 1024
L = 16
NC = 2
NS = 16
NW = NC * NS
ROWS_PER_W = N_ROWS // NW
R = 8
N_CHUNKS = ROWS_PER_W // R


def _gather_phase(in_buf, out_buf, idx_buf, n_groups, stride):
    for j in range(n_groups):
        idx0 = idx_buf[pl.ds(j * L, L)]

        def row_body(r, rowv, j=j, idx0=idx0):
            vals = plsc.load_gather(in_buf, [rowv, idx0])
            out_buf[r, pl.ds(j * L, L)] = vals
            return rowv + 1

        plsc.parallel_loop(0, R, 1, unroll=4, carry=jnp.zeros((L,), jnp.int32))(row_body)


def _body(cont_hbm, bin_hbm, cmask_hbm, bmask_hbm, out_c_hbm, out_b_hbm,
          cidx, bidx,
          cin0, cin1, cout0, cout1, bin0, bin1, bout0, bout1,
          sci0, sci1, sbi0, sbi1, sco0, sco1, sbo0, sbo1):
    cin = (cin0, cin1)
    cout = (cout0, cout1)
    bins = (bin0, bin1)
    bout = (bout0, bout1)
    sci = (sci0, sci1)
    sbi = (sbi0, sbi1)
    sco = (sco0, sco1)
    sbo = (sbo0, sbo1)

    wid = lax.axis_index("c") * NS + lax.axis_index("s")
    base0 = wid * ROWS_PER_W
    pltpu.sync_copy(cmask_hbm, cidx)
    pltpu.sync_copy(bmask_hbm, bidx)

    def start_in(chunk, b):
        base = base0 + chunk * R
        pltpu.async_copy(
            cont_hbm.at[pl.ds(base, R), pl.ds(0, C_TRUNC)], cin[b], sci[b])
        pltpu.async_copy(
            bin_hbm.at[pl.ds(base, R), pl.ds(0, B_TRUNC)], bins[b], sbi[b])

    start_in(0, 0)

    def pair_body(g, carry):
        for b in (0, 1):
            chunk = 2 * g + b
            base = base0 + chunk * R

            @pl.when(chunk + 1 < N_CHUNKS)
            def _(b=b, chunk=chunk):
                start_in(chunk + 1, 1 - b)

            @pl.when(chunk >= 2)
            def _(b=b, chunk=chunk):
                pb = base0 + (chunk - 2) * R
                pltpu.make_async_copy(
                    cout[b], out_c_hbm.at[pl.ds(pb, R), :], sco[b]).wait()
                pltpu.make_async_copy(
                    bout[b], out_b_hbm.at[pl.ds(pb, R), :], sbo[b]).wait()

            pltpu.make_async_copy(
                cont_hbm.at[pl.ds(base, R), pl.ds(0, C_TRUNC)],
                cin[b], sci[b]).wait()
            _gather_phase(cin[b], cout[b], cidx, C_OUT // L, C_TRUNC)
            pltpu.async_copy(cout[b], out_c_hbm.at[pl.ds(base, R), :], sco[b])

            pltpu.make_async_copy(
                bin_hbm.at[pl.ds(base, R), pl.ds(0, B_TRUNC)],
                bins[b], sbi[b]).wait()
            _gather_phase(bins[b], bout[b], bidx, B_OUT // L, B_TRUNC)
            pltpu.async_copy(bout[b], out_b_hbm.at[pl.ds(base, R), :], sbo[b])
        return carry

    lax.fori_loop(0, N_CHUNKS // 2, pair_body, 0)

    for b in (0, 1):
        pb = base0 + (N_CHUNKS - 2 + b) * R
        pltpu.make_async_copy(
            cout[b], out_c_hbm.at[pl.ds(pb, R), :], sco[b]).wait()
        pltpu.make_async_copy(
            bout[b], out_b_hbm.at[pl.ds(pb, R), :], sbo[b]).wait()


def kernel(continuous, binary, continuous_mask, binary_mask):
    mesh = plsc.VectorSubcoreMesh(core_axis_name="c", subcore_axis_name="s")
    k = pl.kernel(
        _body,
        out_type=(
            jax.ShapeDtypeStruct((N_ROWS, C_OUT), jnp.float32),
            jax.ShapeDtypeStruct((N_ROWS, B_OUT), jnp.float32),
        ),
        mesh=mesh,
        compiler_params=pltpu.CompilerParams(
            use_tc_tiling_on_sc=True, needs_layout_passes=False),
        scratch_types=[
            pltpu.VMEM((C_OUT,), jnp.int32),
            pltpu.VMEM((B_OUT,), jnp.int32),
            pltpu.VMEM((R, C_TRUNC), jnp.float32),
            pltpu.VMEM((R, C_TRUNC), jnp.float32),
            pltpu.VMEM((R, C_OUT), jnp.float32),
            pltpu.VMEM((R, C_OUT), jnp.float32),
            pltpu.VMEM((R, B_TRUNC), jnp.float32),
            pltpu.VMEM((R, B_TRUNC), jnp.float32),
            pltpu.VMEM((R, B_OUT), jnp.float32),
            pltpu.VMEM((R, B_OUT), jnp.float32),
            pltpu.SemaphoreType.DMA,
            pltpu.SemaphoreType.DMA,
            pltpu.SemaphoreType.DMA,
            pltpu.SemaphoreType.DMA,
            pltpu.SemaphoreType.DMA,
            pltpu.SemaphoreType.DMA,
            pltpu.SemaphoreType.DMA,
            pltpu.SemaphoreType.DMA,
        ],
    )
    return k(continuous, binary, continuous_mask, binary_mask)


# speedup vs baseline: 2.9703x; 1.0619x over previous
import jax
import jax.numpy as jnp
from jax import lax
from jax.experimental import pallas as pl
from jax.experimental.pallas import tpu as pltpu
from jax.experimental.pallas import tpu_sc as plsc

N_ROWS = 16384
C_TRUNC = 1024
B_TRUNC = 2048
C_OUT = 512
B_OUT = 1024
L = 16
NC = 2
NS = 16
NW = NC * NS
ROWS_PER_W = N_ROWS // NW
R = 16
N_CHUNKS = ROWS_PER_W // R


def _gather_phase(in_buf, out_buf, idx_buf, n_groups):
    for j in range(n_groups):
        idx0 = idx_buf[pl.ds(j * L, L)]

        def row_body(r, rowv, j=j, idx0=idx0):
            vals = plsc.load_gather(in_buf, [rowv, idx0])
            out_buf[r, pl.ds(j * L, L)] = vals
            return rowv + 1

        plsc.parallel_loop(0, R, 1, unroll=4,
                           carry=jnp.zeros((L,), jnp.int32))(row_body)


def _body(cont_hbm, bin_hbm, cmask_hbm, bmask_hbm, out_c_hbm, out_b_hbm,
          cidx, bidx,
          cin0, cin1, bin0, bin1, cout, bout,
          sci0, sci1, sbi0, sbi1, sco, sbo):
    cin = (cin0, cin1)
    bins = (bin0, bin1)
    sci = (sci0, sci1)
    sbi = (sbi0, sbi1)

    wid = lax.axis_index("c") * NS + lax.axis_index("s")
    base0 = wid * ROWS_PER_W
    pltpu.sync_copy(cmask_hbm, cidx)
    pltpu.sync_copy(bmask_hbm, bidx)

    def start_in(chunk, b):
        base = base0 + chunk * R
        pltpu.async_copy(
            cont_hbm.at[pl.ds(base, R), pl.ds(0, C_TRUNC)], cin[b], sci[b])
        pltpu.async_copy(
            bin_hbm.at[pl.ds(base, R), pl.ds(0, B_TRUNC)], bins[b], sbi[b])

    start_in(0, 0)

    def pair_body(g, carry):
        for b in (0, 1):
            chunk = 2 * g + b
            base = base0 + chunk * R

            @pl.when(chunk + 1 < N_CHUNKS)
            def _(b=b, chunk=chunk):
                start_in(chunk + 1, 1 - b)

            pltpu.make_async_copy(
                cont_hbm.at[pl.ds(base, R), pl.ds(0, C_TRUNC)],
                cin[b], sci[b]).wait()

            @pl.when(chunk >= 1)
            def _(chunk=chunk):
                pb = base0 + (chunk - 1) * R
                pltpu.make_async_copy(
                    cout, out_c_hbm.at[pl.ds(pb, R), :], sco).wait()

            _gather_phase(cin[b], cout, cidx, C_OUT // L)
            pltpu.async_copy(cout, out_c_hbm.at[pl.ds(base, R), :], sco)

            pltpu.make_async_copy(
                bin_hbm.at[pl.ds(base, R), pl.ds(0, B_TRUNC)],
                bins[b], sbi[b]).wait()

            @pl.when(chunk >= 1)
            def _(chunk=chunk):
                pb = base0 + (chunk - 1) * R
                pltpu.make_async_copy(
                    bout, out_b_hbm.at[pl.ds(pb, R), :], sbo).wait()

            _gather_phase(bins[b], bout, bidx, B_OUT // L)
            pltpu.async_copy(bout, out_b_hbm.at[pl.ds(base, R), :], sbo)
        return carry

    lax.fori_loop(0, N_CHUNKS // 2, pair_body, 0)

    pb = base0 + (N_CHUNKS - 1) * R
    pltpu.make_async_copy(cout, out_c_hbm.at[pl.ds(pb, R), :], sco).wait()
    pltpu.make_async_copy(bout, out_b_hbm.at[pl.ds(pb, R), :], sbo).wait()


def kernel(continuous, binary, continuous_mask, binary_mask):
    mesh = plsc.VectorSubcoreMesh(core_axis_name="c", subcore_axis_name="s")
    k = pl.kernel(
        _body,
        out_type=(
            jax.ShapeDtypeStruct((N_ROWS, C_OUT), jnp.float32),
            jax.ShapeDtypeStruct((N_ROWS, B_OUT), jnp.float32),
        ),
        mesh=mesh,
        compiler_params=pltpu.CompilerParams(
            use_tc_tiling_on_sc=True, needs_layout_passes=False),
        scratch_types=[
            pltpu.VMEM((C_OUT,), jnp.int32),
            pltpu.VMEM((B_OUT,), jnp.int32),
            pltpu.VMEM((R, C_TRUNC), jnp.float32),
            pltpu.VMEM((R, C_TRUNC), jnp.float32),
            pltpu.VMEM((R, B_TRUNC), jnp.float32),
            pltpu.VMEM((R, B_TRUNC), jnp.float32),
            pltpu.VMEM((R, C_OUT), jnp.float32),
            pltpu.VMEM((R, B_OUT), jnp.float32),
            pltpu.SemaphoreType.DMA,
            pltpu.SemaphoreType.DMA,
            pltpu.SemaphoreType.DMA,
            pltpu.SemaphoreType.DMA,
            pltpu.SemaphoreType.DMA,
            pltpu.SemaphoreType.DMA,
        ],
    )
    return k(continuous, binary, continuous_mask, binary_mask)
